# baseline (speedup 1.0000x reference)
import functools

import jax
import jax.numpy as jnp
from jax import lax
from jax.experimental import pallas as pl
from jax.experimental.pallas import tpu as pltpu
from jax.experimental.pallas import tpu_sc as plsc

_CHUNK = 128


@functools.lru_cache(maxsize=None)
def _make_gather(B, DU, DC, DM):
    info = plsc.get_sparse_core_info()
    NC, NS = info.num_cores, info.num_subcores
    NW = NC * NS
    assert B % (NW * _CHUNK) == 0
    BPW = B // NW
    NCH = BPW // _CHUNK
    mesh = plsc.VectorSubcoreMesh(core_axis_name="c", subcore_axis_name="s")

    @functools.partial(
        pl.kernel,
        mesh=mesh,
        compiler_params=pltpu.CompilerParams(use_tc_tiling_on_sc=False),
        out_type=(
            jax.ShapeDtypeStruct((B, DU), jnp.float32),
            jax.ShapeDtypeStruct((B, DC), jnp.float32),
            jax.ShapeDtypeStruct((B, DM), jnp.float32),
        ),
        scratch_types=[
            pltpu.VMEM((NCH, _CHUNK), jnp.int32),
            pltpu.VMEM((NCH, _CHUNK), jnp.int32),
            pltpu.VMEM((NCH, _CHUNK), jnp.int32),
            pltpu.VMEM((BPW, DU), jnp.float32),
            pltpu.VMEM((BPW, DC), jnp.float32),
            pltpu.VMEM((BPW, DM), jnp.float32),
            pltpu.SemaphoreType.DMA,
        ],
    )
    def gather3(uid_hbm, cid_hbm, mid_hbm, ut_hbm, ct_hbm, mt_hbm,
                u_out, c_out, m_out,
                uidx, cidx, midx, urows, crows, mrows, sem):
        wid = lax.axis_index("s") * NC + lax.axis_index("c")
        base = wid * BPW
        pltpu.sync_copy(uid_hbm.at[wid], uidx)
        pltpu.sync_copy(cid_hbm.at[wid], cidx)
        pltpu.sync_copy(mid_hbm.at[wid], midx)
        copies = []
        for k in range(NCH):
            sl = pl.ds(k * _CHUNK, _CHUNK)
            copies.append(pltpu.async_copy(ut_hbm.at[uidx.at[k]], urows.at[sl], sem))
            copies.append(pltpu.async_copy(ct_hbm.at[cidx.at[k]], crows.at[sl], sem))
            copies.append(pltpu.async_copy(mt_hbm.at[midx.at[k]], mrows.at[sl], sem))
        for cp in copies:
            cp.wait()
        pltpu.sync_copy(urows, u_out.at[pl.ds(base, BPW)])
        pltpu.sync_copy(crows, c_out.at[pl.ds(base, BPW)])
        pltpu.sync_copy(mrows, m_out.at[pl.ds(base, BPW)])

    return gather3


def _mlp_body(ue, ce, me, w1u, w1c, w1m, b1, w2t, b2, out):
    h = jnp.dot(ue[...], w1u[...], preferred_element_type=jnp.float32)
    h = h + jnp.dot(ce[...], w1c[...], preferred_element_type=jnp.float32)
    h = h + jnp.dot(me[...], w1m[...], preferred_element_type=jnp.float32)
    h = jnp.maximum(h + b1[...], 0.0)
    out[...] = jnp.dot(h, w2t[...], preferred_element_type=jnp.float32) + b2[...]


@functools.lru_cache(maxsize=None)
def _make_mlp(B, DU, DC, DM, H, DO, BLK):
    grid = (B // BLK,)

    def row_block(d):
        return pl.BlockSpec((BLK, d), lambda i: (i, 0))

    def full_block(r, c):
        return pl.BlockSpec((r, c), lambda i: (0, 0))

    return pl.pallas_call(
        _mlp_body,
        grid=grid,
        in_specs=[
            row_block(DU), row_block(DC), row_block(DM),
            full_block(DU, H), full_block(DC, H), full_block(DM, H),
            full_block(1, H), full_block(H, DO), full_block(1, DO),
        ],
        out_specs=row_block(DO),
        out_shape=jax.ShapeDtypeStruct((B, DO), jnp.float32),
    )


def kernel(user_id, category_id, month, user_table, cat_table, month_table, W1, b1, W2, b2):
    B = user_id.shape[0]
    DU = user_table.shape[1]
    DC = cat_table.shape[1]
    DM = month_table.shape[1]
    H = W1.shape[0]
    DO = W2.shape[0]

    info = plsc.get_sparse_core_info()
    NW = info.num_cores * info.num_subcores
    idx_shape = (NW, B // (NW * _CHUNK), _CHUNK)
    uid = user_id.astype(jnp.int32).reshape(idx_shape)
    cid = category_id.astype(jnp.int32).reshape(idx_shape)
    mid = month.astype(jnp.int32).reshape(idx_shape)

    u_emb, c_emb, m_emb = _make_gather(B, DU, DC, DM)(
        uid, cid, mid, user_table, cat_table, month_table)

    W1t = W1.T
    w1u = W1t[:DU]
    w1c = W1t[DU:DU + DC]
    w1m = W1t[DU + DC:]
    mlp = _make_mlp(B, DU, DC, DM, H, DO, 2048)
    return mlp(u_emb, c_emb, m_emb, w1u, w1c, w1m,
               b1.reshape(1, H), W2.T, b2.reshape(1, DO))


# speedup vs baseline: 1.5466x; 1.5466x over previous
import functools

import jax
import jax.numpy as jnp
from jax import lax
from jax.experimental import pallas as pl
from jax.experimental.pallas import tpu as pltpu
from jax.experimental.pallas import tpu_sc as plsc

_C = 256


@functools.lru_cache(maxsize=None)
def _make_gather(B, DU, DC, DM):
    info = plsc.get_sparse_core_info()
    NC, NS = info.num_cores, info.num_subcores
    NW = NC * NS
    assert B % NW == 0
    BPW = B // NW
    assert BPW % _C == 0
    mesh = plsc.VectorSubcoreMesh(core_axis_name="c", subcore_axis_name="s")

    @functools.partial(
        pl.kernel,
        mesh=mesh,
        out_type=(
            jax.ShapeDtypeStruct((B, DU), jnp.float32),
            jax.ShapeDtypeStruct((B, DC), jnp.float32),
            jax.ShapeDtypeStruct((B, DM), jnp.float32),
        ),
        scratch_types=[
            pltpu.VMEM((BPW,), jnp.int32),
            pltpu.VMEM((BPW,), jnp.int32),
            pltpu.VMEM((BPW,), jnp.int32),
            pltpu.VMEM((_C, DU), jnp.float32),
            pltpu.VMEM((_C, DC), jnp.float32),
            pltpu.VMEM((_C, DM), jnp.float32),
            pltpu.SemaphoreType.DMA,
        ],
    )
    def gather3(uid_hbm, cid_hbm, mid_hbm, ut_hbm, ct_hbm, mt_hbm,
                u_out, c_out, m_out,
                uidx_s, cidx_s, midx_s,
                urows, crows, mrows, sem):
        wid = lax.axis_index("s") * NC + lax.axis_index("c")
        base = wid * BPW
        pltpu.sync_copy(uid_hbm.at[pl.ds(base, BPW)], uidx_s)
        pltpu.sync_copy(cid_hbm.at[pl.ds(base, BPW)], cidx_s)
        pltpu.sync_copy(mid_hbm.at[pl.ds(base, BPW)], midx_s)

        for ch in range(BPW // _C):
            off = ch * _C

            def body(g, _):
                uvec = uidx_s[pl.ds(off + g * 16, 16)]
                cvec = cidx_s[pl.ds(off + g * 16, 16)]
                mvec = midx_s[pl.ds(off + g * 16, 16)]
                for j in range(16):
                    k = g * 16 + j
                    pltpu.async_copy(ut_hbm.at[uvec[j]], urows.at[k], sem)
                    pltpu.async_copy(ct_hbm.at[cvec[j]], crows.at[k], sem)
                    pltpu.async_copy(mt_hbm.at[mvec[j]], mrows.at[k], sem)
                return ()

            lax.fori_loop(0, _C // 16, body, ())
            pltpu.make_async_copy(u_out.at[pl.ds(0, _C)], urows, sem).wait()
            pltpu.make_async_copy(c_out.at[pl.ds(0, _C)], crows, sem).wait()
            pltpu.make_async_copy(m_out.at[pl.ds(0, _C)], mrows, sem).wait()
            pltpu.sync_copy(urows, u_out.at[pl.ds(base + off, _C)])
            pltpu.sync_copy(crows, c_out.at[pl.ds(base + off, _C)])
            pltpu.sync_copy(mrows, m_out.at[pl.ds(base + off, _C)])

    return gather3


def _mlp_body(ue, ce, me, w1u, w1c, w1m, b1, w2t, b2, out):
    h = jnp.dot(ue[...], w1u[...], preferred_element_type=jnp.float32)
    h = h + jnp.dot(ce[...], w1c[...], preferred_element_type=jnp.float32)
    h = h + jnp.dot(me[...], w1m[...], preferred_element_type=jnp.float32)
    h = jnp.maximum(h + b1[...], 0.0)
    out[...] = jnp.dot(h, w2t[...], preferred_element_type=jnp.float32) + b2[...]


@functools.lru_cache(maxsize=None)
def _make_mlp(B, DU, DC, DM, H, DO, BLK):
    grid = (B // BLK,)

    def row_block(d):
        return pl.BlockSpec((BLK, d), lambda i: (i, 0))

    def full_block(r, c):
        return pl.BlockSpec((r, c), lambda i: (0, 0))

    return pl.pallas_call(
        _mlp_body,
        grid=grid,
        in_specs=[
            row_block(DU), row_block(DC), row_block(DM),
            full_block(DU, H), full_block(DC, H), full_block(DM, H),
            full_block(1, H), full_block(H, DO), full_block(1, DO),
        ],
        out_specs=row_block(DO),
        out_shape=jax.ShapeDtypeStruct((B, DO), jnp.float32),
    )


def kernel(user_id, category_id, month, user_table, cat_table, month_table, W1, b1, W2, b2):
    B = user_id.shape[0]
    DU = user_table.shape[1]
    DC = cat_table.shape[1]
    DM = month_table.shape[1]
    H = W1.shape[0]
    DO = W2.shape[0]

    uid = user_id.astype(jnp.int32)
    cid = category_id.astype(jnp.int32)
    mid = month.astype(jnp.int32)

    u_emb, c_emb, m_emb = _make_gather(B, DU, DC, DM)(
        uid, cid, mid, user_table, cat_table, month_table)

    W1t = W1.T
    w1u = W1t[:DU]
    w1c = W1t[DU:DU + DC]
    w1m = W1t[DU + DC:]
    mlp = _make_mlp(B, DU, DC, DM, H, DO, 2048)
    return mlp(u_emb, c_emb, m_emb, w1u, w1c, w1m,
               b1.reshape(1, H), W2.T, b2.reshape(1, DO))


# speedup vs baseline: 1.8395x; 1.1894x over previous
import functools

import jax
import jax.numpy as jnp
from jax import lax
from jax.experimental import pallas as pl
from jax.experimental.pallas import tpu as pltpu
from jax.experimental.pallas import tpu_sc as plsc

_C = 256
_TBLK = 8192


def _transpose_body(src, dst):
    dst[...] = src[...].T


@functools.lru_cache(maxsize=None)
def _make_transpose(D, V):
    grid = (pl.cdiv(V, _TBLK),)
    return pl.pallas_call(
        _transpose_body,
        grid=grid,
        in_specs=[pl.BlockSpec((D, _TBLK), lambda i: (0, i))],
        out_specs=pl.BlockSpec((_TBLK, D), lambda i: (i, 0)),
        out_shape=jax.ShapeDtypeStruct((V, D), jnp.float32),
    )


@functools.lru_cache(maxsize=None)
def _make_gather(B, DU, DC, DM):
    info = plsc.get_sparse_core_info()
    NC, NS = info.num_cores, info.num_subcores
    NW = NC * NS
    assert B % NW == 0
    BPW = B // NW
    assert BPW % _C == 0
    mesh = plsc.VectorSubcoreMesh(core_axis_name="c", subcore_axis_name="s")

    @functools.partial(
        pl.kernel,
        mesh=mesh,
        out_type=(
            jax.ShapeDtypeStruct((B, DU), jnp.float32),
            jax.ShapeDtypeStruct((B, DC), jnp.float32),
            jax.ShapeDtypeStruct((B, DM), jnp.float32),
        ),
        scratch_types=[
            pltpu.VMEM((BPW,), jnp.int32),
            pltpu.VMEM((BPW,), jnp.int32),
            pltpu.VMEM((BPW,), jnp.int32),
            pltpu.VMEM((_C, DU), jnp.float32),
            pltpu.VMEM((_C, DC), jnp.float32),
            pltpu.VMEM((_C, DM), jnp.float32),
            pltpu.SemaphoreType.DMA,
        ],
    )
    def gather3(uid_hbm, cid_hbm, mid_hbm, ut_hbm, ct_hbm, mt_hbm,
                u_out, c_out, m_out,
                uidx_s, cidx_s, midx_s,
                urows, crows, mrows, sem):
        wid = lax.axis_index("s") * NC + lax.axis_index("c")
        base = wid * BPW
        pltpu.sync_copy(uid_hbm.at[pl.ds(base, BPW)], uidx_s)
        pltpu.sync_copy(cid_hbm.at[pl.ds(base, BPW)], cidx_s)
        pltpu.sync_copy(mid_hbm.at[pl.ds(base, BPW)], midx_s)

        for ch in range(BPW // _C):
            off = ch * _C

            def body(g, _):
                uvec = uidx_s[pl.ds(off + g * 16, 16)]
                cvec = cidx_s[pl.ds(off + g * 16, 16)]
                mvec = midx_s[pl.ds(off + g * 16, 16)]
                for j in range(16):
                    k = g * 16 + j
                    pltpu.async_copy(ut_hbm.at[uvec[j]], urows.at[k], sem)
                    pltpu.async_copy(ct_hbm.at[cvec[j]], crows.at[k], sem)
                    pltpu.async_copy(mt_hbm.at[mvec[j]], mrows.at[k], sem)
                return ()

            lax.fori_loop(0, _C // 16, body, ())
            pltpu.make_async_copy(u_out.at[pl.ds(0, _C)], urows, sem).wait()
            pltpu.make_async_copy(c_out.at[pl.ds(0, _C)], crows, sem).wait()
            pltpu.make_async_copy(m_out.at[pl.ds(0, _C)], mrows, sem).wait()
            pltpu.sync_copy(urows, u_out.at[pl.ds(base + off, _C)])
            pltpu.sync_copy(crows, c_out.at[pl.ds(base + off, _C)])
            pltpu.sync_copy(mrows, m_out.at[pl.ds(base + off, _C)])

    return gather3


def _mlp_body(ue, ce, me, w1u, w1c, w1m, b1, w2t, b2, out):
    h = jnp.dot(ue[...], w1u[...], preferred_element_type=jnp.float32)
    h = h + jnp.dot(ce[...], w1c[...], preferred_element_type=jnp.float32)
    h = h + jnp.dot(me[...], w1m[...], preferred_element_type=jnp.float32)
    h = jnp.maximum(h + b1[...], 0.0)
    out[...] = jnp.dot(h, w2t[...], preferred_element_type=jnp.float32) + b2[...]


@functools.lru_cache(maxsize=None)
def _make_mlp(B, DU, DC, DM, H, DO, BLK):
    grid = (B // BLK,)

    def row_block(d):
        return pl.BlockSpec((BLK, d), lambda i: (i, 0))

    def full_block(r, c):
        return pl.BlockSpec((r, c), lambda i: (0, 0))

    return pl.pallas_call(
        _mlp_body,
        grid=grid,
        in_specs=[
            row_block(DU), row_block(DC), row_block(DM),
            full_block(DU, H), full_block(DC, H), full_block(DM, H),
            full_block(1, H), full_block(H, DO), full_block(1, DO),
        ],
        out_specs=row_block(DO),
        out_shape=jax.ShapeDtypeStruct((B, DO), jnp.float32),
    )


def kernel(user_id, category_id, month, user_table, cat_table, month_table, W1, b1, W2, b2):
    B = user_id.shape[0]
    DU = user_table.shape[1]
    DC = cat_table.shape[1]
    DM = month_table.shape[1]
    H = W1.shape[0]
    DO = W2.shape[0]

    uid = user_id.astype(jnp.int32)
    cid = category_id.astype(jnp.int32)
    mid = month.astype(jnp.int32)

    ut_rm = _make_transpose(DU, user_table.shape[0])(user_table.T)
    u_emb, c_emb, m_emb = _make_gather(B, DU, DC, DM)(
        uid, cid, mid, ut_rm, cat_table, month_table)

    W1t = W1.T
    w1u = W1t[:DU]
    w1c = W1t[DU:DU + DC]
    w1m = W1t[DU + DC:]
    mlp = _make_mlp(B, DU, DC, DM, H, DO, 2048)
    return mlp(u_emb, c_emb, m_emb, w1u, w1c, w1m,
               b1.reshape(1, H), W2.T, b2.reshape(1, DO))


# speedup vs baseline: 2.3229x; 1.2628x over previous
import functools

import jax
import jax.numpy as jnp
from jax import lax
from jax.experimental import pallas as pl
from jax.experimental.pallas import tpu as pltpu
from jax.experimental.pallas import tpu_sc as plsc

_TBLK = 16384
_C = 256
_S = 128


def _pack2_body(src, dst):
    sv = src[...]
    eye = jnp.eye(sv.shape[0], dtype=sv.dtype)
    t = lax.dot_general(sv, eye, (((0,), (0,)), ((), ())),
                        preferred_element_type=jnp.float32)
    h = t.shape[0] // 2
    dst[...] = jnp.concatenate([t[:h], t[h:]], axis=1)


@functools.lru_cache(maxsize=None)
def _make_pack2(D, V, blk):
    nb = pl.cdiv(V, blk)
    grid = (nb,)
    return pl.pallas_call(
        _pack2_body,
        grid=grid,
        in_specs=[pl.BlockSpec((D, blk), lambda i: (0, i))],
        out_specs=pl.BlockSpec((blk // 2, 2 * D), lambda i: (i, 0)),
        out_shape=jax.ShapeDtypeStruct((nb * blk // 2, 2 * D), jnp.float32),
    )


def _pack4_body(src, dst):
    t = src[...].T
    q = t.shape[0] // 4
    dst[...] = jnp.concatenate([t[:q], t[q:2 * q], t[2 * q:3 * q], t[3 * q:]], axis=1)


@functools.lru_cache(maxsize=None)
def _make_pack4(D, V):
    return pl.pallas_call(
        _pack4_body,
        in_specs=[pl.BlockSpec((D, V), lambda: (0, 0))],
        out_specs=pl.BlockSpec((V // 4, 4 * D), lambda: (0, 0)),
        out_shape=jax.ShapeDtypeStruct((V // 4, 4 * D), jnp.float32),
    )


@functools.lru_cache(maxsize=None)
def _make_gather(B):
    info = plsc.get_sparse_core_info()
    NC, NS = info.num_cores, info.num_subcores
    NW = NC * NS
    assert B % NW == 0
    BPW = B // NW
    assert BPW % _C == 0 and _C % _S == 0
    NLISTS = BPW // _S
    mesh = plsc.VectorSubcoreMesh(core_axis_name="c", subcore_axis_name="s")

    @functools.partial(
        pl.kernel,
        mesh=mesh,
        out_type=(
            jax.ShapeDtypeStruct((B, 128), jnp.float32),
            jax.ShapeDtypeStruct((B, 128), jnp.float32),
        ),
        scratch_types=[
            pltpu.VMEM((BPW,), jnp.int32),
            pltpu.VMEM((BPW,), jnp.int32),
            pltpu.VMEM((NLISTS, _S), jnp.int32),
            pltpu.VMEM((NLISTS, _S), jnp.int32),
            pltpu.VMEM((_C, 128), jnp.float32),
            pltpu.VMEM((_C, 128), jnp.float32),
            pltpu.SemaphoreType.DMA,
        ],
    )
    def gather2(uid_hbm, cid_hbm, utp_hbm, ctp_hbm,
                u_out, c_out,
                uidx, cidx, ulist, clist, uwide, cwide, sem):
        wid = lax.axis_index("s") * NC + lax.axis_index("c")
        base = wid * BPW
        pltpu.sync_copy(uid_hbm.at[pl.ds(base, BPW)], uidx)
        pltpu.sync_copy(cid_hbm.at[pl.ds(base, BPW)], cidx)
        for g in range(BPW // 16):
            row, col = g // (_S // 16), (g % (_S // 16)) * 16
            sl = pl.ds(g * 16, 16)
            uv = uidx[sl]
            ulist[row, pl.ds(col, 16)] = lax.bitwise_or(
                lax.shift_left(lax.shift_right_logical(uv, 13), 12),
                lax.bitwise_and(uv, 4095))
            clist[row, pl.ds(col, 16)] = lax.bitwise_and(cidx[sl], 255)
        lists_per_chunk = _C // _S
        for ch in range(BPW // _C):
            copies = []
            for j in range(lists_per_chunk):
                k = ch * lists_per_chunk + j
                dst = pl.ds(j * _S, _S)
                copies.append(
                    pltpu.async_copy(utp_hbm.at[ulist.at[k]], uwide.at[dst], sem))
                copies.append(
                    pltpu.async_copy(ctp_hbm.at[clist.at[k]], cwide.at[dst], sem))
            for cp in copies:
                cp.wait()
            off = ch * _C
            pltpu.sync_copy(uwide, u_out.at[pl.ds(base + off, _C)])
            pltpu.sync_copy(cwide, c_out.at[pl.ds(base + off, _C)])

    return gather2


def _mlp_body(uw, cw, uid, cid, mid, mt, w1u, w1c, w1m, b1, w2t, b2, out):
    blk = uw.shape[0]
    uwv = uw[...]
    par = jnp.bitwise_and(lax.shift_right_logical(uid[...], 12), 1).astype(jnp.float32)
    ue = uwv[:, :64] * (1.0 - par) + uwv[:, 64:] * par
    cwv = cw[...]
    q = lax.shift_right_logical(cid[...], 8)
    ce = jnp.zeros((blk, 32), jnp.float32)
    for j in range(4):
        mj = (q == j).astype(jnp.float32)
        ce = ce + cwv[:, 32 * j:32 * j + 32] * mj
    moh = (mid[...] == lax.broadcasted_iota(jnp.int32, (blk, 12), 1)
           ).astype(jnp.float32)
    me = jnp.dot(moh, mt[...], preferred_element_type=jnp.float32)
    h = jnp.dot(ue, w1u[...], preferred_element_type=jnp.float32)
    h = h + jnp.dot(ce, w1c[...], preferred_element_type=jnp.float32)
    h = h + jnp.dot(me, w1m[...], preferred_element_type=jnp.float32)
    h = jnp.maximum(h + b1[...], 0.0)
    out[...] = jnp.dot(h, w2t[...], preferred_element_type=jnp.float32) + b2[...]


@functools.lru_cache(maxsize=None)
def _make_mlp(B, H, DO, BLK):
    grid = (B // BLK,)

    def row_block(d):
        return pl.BlockSpec((BLK, d), lambda i: (i, 0))

    def full_block(r, c):
        return pl.BlockSpec((r, c), lambda i: (0, 0))

    return pl.pallas_call(
        _mlp_body,
        grid=grid,
        in_specs=[
            row_block(128), row_block(128),
            row_block(1), row_block(1), row_block(1),
            full_block(12, 16),
            full_block(64, H), full_block(32, H), full_block(16, H),
            full_block(1, H), full_block(H, DO), full_block(1, DO),
        ],
        out_specs=row_block(DO),
        out_shape=jax.ShapeDtypeStruct((B, DO), jnp.float32),
    )


def kernel(user_id, category_id, month, user_table, cat_table, month_table, W1, b1, W2, b2):
    B = user_id.shape[0]
    DU = user_table.shape[1]
    DC = cat_table.shape[1]
    H = W1.shape[0]
    DO = W2.shape[0]

    uid = user_id.astype(jnp.int32)
    cid = category_id.astype(jnp.int32)
    mid = month.astype(jnp.int32)

    utp = _make_pack2(DU, user_table.shape[0], _TBLK)(user_table.T)
    ctt = jnp.pad(cat_table.T, ((0, 0), (0, 1024 - cat_table.shape[0])))
    ctp = _make_pack4(DC, 1024)(ctt)

    uw, cw = _make_gather(B)(uid, cid, utp, ctp)

    W1t = W1.T
    w1u = W1t[:DU]
    w1c = W1t[DU:DU + DC]
    w1m = W1t[DU + DC:]
    mlp = _make_mlp(B, H, DO, 2048)
    return mlp(uw, cw, uid.reshape(B, 1), cid.reshape(B, 1), mid.reshape(B, 1),
               month_table, w1u, w1c, w1m,
               b1.reshape(1, H), W2.T, b2.reshape(1, DO))


# speedup vs baseline: 2.9536x; 1.2715x over previous
import functools

import jax
import jax.numpy as jnp
from jax import lax
from jax.experimental import pallas as pl
from jax.experimental.pallas import tpu as pltpu
from jax.experimental.pallas import tpu_sc as plsc

_TBLK = 16384
_C = 256
_S = 128


def _pack2_body(src, dst):
    t = src[...].T
    h = t.shape[0] // 2
    a16 = t[:h].astype(jnp.bfloat16)
    b16 = t[h:].astype(jnp.bfloat16)
    au = lax.bitcast_convert_type(a16, jnp.uint16).astype(jnp.uint32)
    bu = lax.bitcast_convert_type(b16, jnp.uint16).astype(jnp.uint32)
    w = lax.bitwise_or(au, lax.shift_left(bu, jnp.uint32(16))).astype(jnp.int32)
    q = w.shape[0] // 2
    dst[...] = jnp.concatenate([w[:q], w[q:]], axis=1)


@functools.lru_cache(maxsize=None)
def _make_pack2(D, V, blk):
    nb = pl.cdiv(V, blk)
    grid = (nb,)
    return pl.pallas_call(
        _pack2_body,
        grid=grid,
        in_specs=[pl.BlockSpec((D, blk), lambda i: (0, i))],
        out_specs=pl.BlockSpec((blk // 4, 2 * D), lambda i: (i, 0)),
        out_shape=jax.ShapeDtypeStruct((nb * blk // 4, 2 * D), jnp.int32),
    )


def _pack4_body(src, dst):
    t = src[...].T
    q = t.shape[0] // 4
    dst[...] = jnp.concatenate([t[:q], t[q:2 * q], t[2 * q:3 * q], t[3 * q:]], axis=1)


@functools.lru_cache(maxsize=None)
def _make_pack4(D, V):
    return pl.pallas_call(
        _pack4_body,
        in_specs=[pl.BlockSpec((D, V), lambda: (0, 0))],
        out_specs=pl.BlockSpec((V // 4, 4 * D), lambda: (0, 0)),
        out_shape=jax.ShapeDtypeStruct((V // 4, 4 * D), jnp.float32),
    )


@functools.lru_cache(maxsize=None)
def _make_gather(B):
    info = plsc.get_sparse_core_info()
    NC, NS = info.num_cores, info.num_subcores
    NW = NC * NS
    assert B % NW == 0
    BPW = B // NW
    assert BPW % _C == 0 and _C % _S == 0
    NLISTS = BPW // _S
    mesh = plsc.VectorSubcoreMesh(core_axis_name="c", subcore_axis_name="s")

    @functools.partial(
        pl.kernel,
        mesh=mesh,
        out_type=(
            jax.ShapeDtypeStruct((B, 128), jnp.int32),
            jax.ShapeDtypeStruct((B, 128), jnp.float32),
        ),
        scratch_types=[
            pltpu.VMEM((BPW,), jnp.int32),
            pltpu.VMEM((BPW,), jnp.int32),
            pltpu.VMEM((NLISTS, _S), jnp.int32),
            pltpu.VMEM((NLISTS, _S), jnp.int32),
            pltpu.VMEM((_C, 128), jnp.int32),
            pltpu.VMEM((_C, 128), jnp.float32),
            pltpu.SemaphoreType.DMA,
        ],
    )
    def gather2(uid_hbm, cid_hbm, utp_hbm, ctp_hbm,
                u_out, c_out,
                uidx, cidx, ulist, clist, uwide, cwide, sem):
        wid = lax.axis_index("s") * NC + lax.axis_index("c")
        base = wid * BPW
        pltpu.sync_copy(uid_hbm.at[pl.ds(base, BPW)], uidx)
        pltpu.sync_copy(cid_hbm.at[pl.ds(base, BPW)], cidx)
        for g in range(BPW // 16):
            row, col = g // (_S // 16), (g % (_S // 16)) * 16
            sl = pl.ds(g * 16, 16)
            uv = uidx[sl]
            ulist[row, pl.ds(col, 16)] = lax.bitwise_or(
                lax.shift_left(lax.shift_right_logical(uv, 14), 12),
                lax.bitwise_and(uv, 4095))
            clist[row, pl.ds(col, 16)] = lax.bitwise_and(cidx[sl], 255)
        lists_per_chunk = _C // _S
        for ch in range(BPW // _C):
            copies = []
            for j in range(lists_per_chunk):
                k = ch * lists_per_chunk + j
                dst = pl.ds(j * _S, _S)
                copies.append(
                    pltpu.async_copy(utp_hbm.at[ulist.at[k]], uwide.at[dst], sem))
                copies.append(
                    pltpu.async_copy(ctp_hbm.at[clist.at[k]], cwide.at[dst], sem))
            for cp in copies:
                cp.wait()
            off = ch * _C
            pltpu.sync_copy(uwide, u_out.at[pl.ds(base + off, _C)])
            pltpu.sync_copy(cwide, c_out.at[pl.ds(base + off, _C)])

    return gather2


def _mlp_body(uw, cw, uid, cid, mid, mt, w1u, w1c, w1m, b1, w2t, b2, out):
    blk = uw.shape[0]
    uwv = uw[...]
    uidv = uid[...]
    j2 = jnp.bitwise_and(lax.shift_right_logical(uidv, jnp.int32(12)), 1) == 1
    w64 = jnp.where(j2, uwv[:, 64:], uwv[:, :64])
    lo = lax.bitcast_convert_type(lax.shift_left(w64, jnp.int32(16)), jnp.float32)
    hi = lax.bitcast_convert_type(
        lax.bitwise_and(w64, jnp.int32(-65536)), jnp.float32)
    j1 = jnp.bitwise_and(lax.shift_right_logical(uidv, jnp.int32(13)), 1) == 1
    ue = jnp.where(j1, hi, lo)
    cwv = cw[...]
    q = lax.shift_right_logical(cid[...], 8)
    ce = jnp.zeros((blk, 32), jnp.float32)
    for j in range(4):
        mj = (q == j).astype(jnp.float32)
        ce = ce + cwv[:, 32 * j:32 * j + 32] * mj
    moh = (mid[...] == lax.broadcasted_iota(jnp.int32, (blk, 12), 1)
           ).astype(jnp.float32)
    me = jnp.dot(moh, mt[...], preferred_element_type=jnp.float32)
    h = jnp.dot(ue, w1u[...], preferred_element_type=jnp.float32)
    h = h + jnp.dot(ce, w1c[...], preferred_element_type=jnp.float32)
    h = h + jnp.dot(me, w1m[...], preferred_element_type=jnp.float32)
    h = jnp.maximum(h + b1[...], 0.0)
    out[...] = jnp.dot(h, w2t[...], preferred_element_type=jnp.float32) + b2[...]


@functools.lru_cache(maxsize=None)
def _make_mlp(B, H, DO, BLK):
    grid = (B // BLK,)

    def row_block(d):
        return pl.BlockSpec((BLK, d), lambda i: (i, 0))

    def full_block(r, c):
        return pl.BlockSpec((r, c), lambda i: (0, 0))

    return pl.pallas_call(
        _mlp_body,
        grid=grid,
        in_specs=[
            row_block(128), row_block(128),
            row_block(1), row_block(1), row_block(1),
            full_block(12, 16),
            full_block(64, H), full_block(32, H), full_block(16, H),
            full_block(1, H), full_block(H, DO), full_block(1, DO),
        ],
        out_specs=row_block(DO),
        out_shape=jax.ShapeDtypeStruct((B, DO), jnp.float32),
    )


def kernel(user_id, category_id, month, user_table, cat_table, month_table, W1, b1, W2, b2):
    B = user_id.shape[0]
    DU = user_table.shape[1]
    DC = cat_table.shape[1]
    H = W1.shape[0]
    DO = W2.shape[0]

    uid = user_id.astype(jnp.int32)
    cid = category_id.astype(jnp.int32)
    mid = month.astype(jnp.int32)

    utp = _make_pack2(DU, user_table.shape[0], _TBLK)(user_table.T)
    ctt = jnp.pad(cat_table.T, ((0, 0), (0, 1024 - cat_table.shape[0])))
    ctp = _make_pack4(DC, 1024)(ctt)

    uw, cw = _make_gather(B)(uid, cid, utp, ctp)

    W1t = W1.T
    w1u = W1t[:DU]
    w1c = W1t[DU:DU + DC]
    w1m = W1t[DU + DC:]
    mlp = _make_mlp(B, H, DO, 2048)
    return mlp(uw, cw, uid.reshape(B, 1), cid.reshape(B, 1), mid.reshape(B, 1),
               month_table, w1u, w1c, w1m,
               b1.reshape(1, H), W2.T, b2.reshape(1, DO))
